# Initial kernel scaffold; baseline (speedup 1.0000x reference)
import jax
import jax.numpy as jnp
from jax.experimental import pallas as pl


def kernel(x):
    raise NotImplementedError("write your pallas kernel here")


# speedup vs baseline: 1.7628x; 1.7628x over previous
import jax
import jax.numpy as jnp
from jax import lax
from jax.experimental import pallas as pl
from jax.experimental.pallas import tpu as pltpu
from jax.experimental.pallas import tpu_sc as plsc

ROWS, N = 64, 8192
K = int(0.05 * N)
NC, NS, L = 2, 16, 16
NW = NC * NS
ROWS_PER_W = ROWS // NW
NVEC = N // L
INT_MIN = jnp.int32(-2147483648)


def _ordered_bits_to_f32(cand_u):
    bits = jnp.where(cand_u < 0, cand_u ^ INT_MIN, ~cand_u)
    return lax.bitcast_convert_type(bits, jnp.float32)


def _body(x_hbm, out_hbm, x_v, sem):
    wid = lax.axis_index("s") * NC + lax.axis_index("c")
    base = wid * ROWS_PER_W
    pltpu.sync_copy(x_hbm.at[pl.ds(base, ROWS_PER_W)], x_v)

    for r in range(ROWS_PER_W):
        def bit_step(b, prefix_u):
            cand_u = prefix_u | lax.shift_left(jnp.int32(1), 31 - b)
            cand_f = _ordered_bits_to_f32(cand_u)

            def count(j, acc):
                xv = x_v[r, pl.ds(j * L, L)]
                return acc + jnp.where(xv >= cand_f, jnp.int32(1),
                                       jnp.int32(0))

            acc = lax.fori_loop(0, NVEC, count,
                                jnp.zeros((L,), jnp.int32))
            cnt = acc[0]
            for e in range(1, L):
                cnt = cnt + acc[e]
            return jnp.where(cnt >= K, cand_u, prefix_u)

        prefix_u = lax.fori_loop(0, 32, bit_step, jnp.int32(0))
        thr_f = _ordered_bits_to_f32(prefix_u)

        def mask_pass(j, carry):
            xv = x_v[r, pl.ds(j * L, L)]
            x_v[r, pl.ds(j * L, L)] = jnp.where(xv >= thr_f, xv,
                                                jnp.float32(0.0))
            return carry

        lax.fori_loop(0, NVEC, mask_pass, jnp.int32(0))

    pltpu.sync_copy(x_v, out_hbm.at[pl.ds(base, ROWS_PER_W)])


@jax.jit
def kernel(x):
    mesh = plsc.VectorSubcoreMesh(
        core_axis_name="c", subcore_axis_name="s",
        num_cores=NC, num_subcores=NS)
    f = pl.kernel(
        _body,
        out_type=jax.ShapeDtypeStruct((ROWS, N), jnp.float32),
        mesh=mesh,
        scratch_types=[
            pltpu.VMEM((ROWS_PER_W, N), jnp.float32),
            pltpu.SemaphoreType.DMA,
        ],
    )
    return f(x)


# speedup vs baseline: 5.7396x; 3.2560x over previous
import jax
import jax.numpy as jnp
from jax import lax
from jax.experimental import pallas as pl
from jax.experimental.pallas import tpu as pltpu
from jax.experimental.pallas import tpu_sc as plsc

ROWS, N = 64, 8192
K = int(0.05 * N)
NC, NS, L = 2, 16, 16
NW = NC * NS
ROWS_PER_W = ROWS // NW
NVEC = N // L
INT_MIN = jnp.int32(-2147483648)


def _ordered_bits_to_f32(cand_u):
    bits = jnp.where(cand_u < 0, cand_u ^ INT_MIN, ~cand_u)
    return lax.bitcast_convert_type(bits, jnp.float32)


def _body(x_hbm, out_hbm, x_v, sem):
    wid = lax.axis_index("s") * NC + lax.axis_index("c")
    base = wid * ROWS_PER_W
    pltpu.sync_copy(x_hbm.at[pl.ds(base, ROWS_PER_W)], x_v)

    UNROLL = 8
    ONE = jnp.full((L,), 1, jnp.int32)
    ZERO = jnp.full((L,), 0, jnp.int32)

    for r in range(ROWS_PER_W):
        def bit_step(b, prefix_u):
            cand_u = prefix_u | lax.shift_left(jnp.int32(1), 31 - b)
            cand_f = _ordered_bits_to_f32(cand_u)

            def count(j, accs):
                new = []
                for u in range(UNROLL):
                    xv = x_v[r, pl.ds((j * UNROLL + u) * L, L)]
                    new.append(accs[u] +
                               jnp.where(xv >= cand_f, ONE, ZERO))
                return tuple(new)

            accs = lax.fori_loop(0, NVEC // UNROLL, count,
                                 tuple(ZERO for _ in range(UNROLL)))
            acc = accs[0]
            for u in range(1, UNROLL):
                acc = acc + accs[u]
            cnt = acc[0]
            for e in range(1, L):
                cnt = cnt + acc[e]
            return jnp.where(cnt >= K, cand_u, prefix_u)

        prefix_u = lax.fori_loop(0, 32, bit_step, jnp.int32(0))
        thr_f = _ordered_bits_to_f32(prefix_u)

        def mask_pass(j, carry):
            for u in range(UNROLL):
                sl = pl.ds((j * UNROLL + u) * L, L)
                xv = x_v[r, sl]
                x_v[r, sl] = jnp.where(xv >= thr_f, xv, jnp.float32(0.0))
            return carry

        lax.fori_loop(0, NVEC // UNROLL, mask_pass, jnp.int32(0))

    pltpu.sync_copy(x_v, out_hbm.at[pl.ds(base, ROWS_PER_W)])


@jax.jit
def kernel(x):
    mesh = plsc.VectorSubcoreMesh(
        core_axis_name="c", subcore_axis_name="s",
        num_cores=NC, num_subcores=NS)
    f = pl.kernel(
        _body,
        out_type=jax.ShapeDtypeStruct((ROWS, N), jnp.float32),
        mesh=mesh,
        scratch_types=[
            pltpu.VMEM((ROWS_PER_W, N), jnp.float32),
            pltpu.SemaphoreType.DMA,
        ],
    )
    return f(x)
